# Initial kernel scaffold; baseline (speedup 1.0000x reference)
import jax
import jax.numpy as jnp
from jax.experimental import pallas as pl


def kernel(input_seq, params):
    raise NotImplementedError("write your pallas kernel here")


# speedup vs baseline: 15.3119x; 15.3119x over previous
import functools

import jax
import jax.numpy as jnp
import numpy as np
from jax.experimental import pallas as pl
from jax.experimental.pallas import tpu as pltpu

SLOTS = 1024
MDIM = 64
MHEADS = 4
TOPK = 32
VOCAB = 8192
EDIM = 512
HDIM = 512
NHATTN = 8
DFF = 2048
B = 4
T = 32

C_LOG = 0
C_RK = 8192
C_WK = 8448
C_WV = 8576
C_ER = 8704
C_AG = 8832
NCOL = 8960
KEY_OFFS = (C_RK, C_RK + 64, C_RK + 128, C_RK + 192, C_WK)


def _scan_body(h_ref, w1_ref, w2_ref, bias_ref, dec_ref, out_ref, mem_ref, p_ref):
    acc = jax.lax.dot_general(
        h_ref[...], w1_ref[...], (((1,), (0,)), ((), ())),
        preferred_element_type=jnp.float32)
    p_ref[...] = (acc + bias_ref[...]).reshape(T, B, NCOL)
    mem_ref[...] = jnp.zeros((B, MDIM, SLOTS), jnp.float32)
    dec = dec_ref[...]

    iota = jax.lax.broadcasted_iota(jnp.int32, (5 * B, SLOTS), 1)

    def step(t, rv):
        pr = p_ref[t] + jax.lax.dot_general(
            rv, w2_ref[...], (((1,), (0,)), ((), ())),
            preferred_element_type=jnp.float32)
        out_ref[t] = pr[:, C_LOG:C_LOG + VOCAB]

        s_rows = []
        for b in range(B):
            kb = jnp.concatenate(
                [pr[b:b + 1, o:o + MDIM] for o in KEY_OFFS], axis=0)
            s_rows.append(jax.lax.dot_general(
                kb, mem_ref[b], (((1,), (0,)), ((), ())),
                preferred_element_type=jnp.float32))
        s_orig = jnp.concatenate(s_rows, axis=0)

        def extract(_, c):
            s, sel = c
            m = jnp.max(s, axis=1, keepdims=True)
            t_idx = jnp.min(jnp.where(s == m, iota, jnp.int32(2 ** 30)),
                            axis=1, keepdims=True)
            oh = iota == t_idx
            return jnp.where(oh, jnp.float32(-1e30), s), jnp.where(oh, 1.0, sel)

        _, sel = jax.lax.fori_loop(
            0, TOPK, extract, (s_orig, jnp.zeros((5 * B, SLOTS), jnp.float32)))

        gmax = jnp.max(s_orig, axis=1, keepdims=True)
        w_un = sel * jnp.exp(s_orig - gmax)
        wf = w_un / jnp.sum(w_un, axis=1, keepdims=True)

        wv_t = jnp.transpose(pr[:, C_WV:C_WV + MDIM])
        er_t = jnp.transpose(jax.nn.sigmoid(pr[:, C_ER:C_ER + MDIM]))
        ag = jax.nn.sigmoid(pr[:, C_AG:C_AG + 1])

        rv_rows = []
        for b in range(B):
            mb = mem_ref[b]
            wr = wf[5 * b:5 * b + MHEADS]
            rb = jax.lax.dot_general(
                wr, mb, (((1,), (1,)), ((), ())),
                preferred_element_type=jnp.float32)
            rv_rows.append(jnp.mean(rb, axis=0, keepdims=True))
            ww = wf[5 * b + MHEADS:5 * b + MHEADS + 1]
            upd = mb * (1.0 - er_t[:, b:b + 1] * ww) \
                + ag[b:b + 1, :] * wv_t[:, b:b + 1] * ww
            mem_ref[b] = upd * dec
        return jnp.concatenate(rv_rows, axis=0)

    jax.lax.fori_loop(0, T, step, jnp.zeros((B, MDIM), jnp.float32))


def _ln(x, g, b):
    m = jnp.mean(x, axis=-1, keepdims=True)
    v = jnp.var(x, axis=-1, keepdims=True)
    return (x - m) / jnp.sqrt(v + 1e-5) * g + b


def _controller_hidden(params, tokens):
    Bq, Tq = tokens.shape
    dh = HDIM // NHATTN
    x = params['embed'][tokens] + params['pos'][None, :Tq, :]
    x = x @ params['W_in']
    mask = jnp.tril(jnp.ones((Tq, Tq), dtype=bool))
    for L in params['layers']:
        h = _ln(x, L['ln1_g'], L['ln1_b'])
        q = (h @ L['Wq']).reshape(Bq, Tq, NHATTN, dh).transpose(0, 2, 1, 3)
        k = (h @ L['Wk']).reshape(Bq, Tq, NHATTN, dh).transpose(0, 2, 1, 3)
        v = (h @ L['Wv']).reshape(Bq, Tq, NHATTN, dh).transpose(0, 2, 1, 3)
        s = jnp.einsum('bhqd,bhkd->bhqk', q, k) / float(np.sqrt(dh))
        s = jnp.where(mask[None, None, :, :], s, -1e9)
        a = jax.nn.softmax(s, axis=-1)
        o = jnp.einsum('bhqk,bhkd->bhqd', a, v).transpose(0, 2, 1, 3).reshape(Bq, Tq, HDIM)
        x = x + o @ L['Wo']
        h2 = _ln(x, L['ln2_g'], L['ln2_b'])
        x = x + jax.nn.gelu(h2 @ L['W1'] + L['b1']) @ L['W2'] + L['b2']
    return _ln(x, params['lnf_g'], params['lnf_b'])


def _build_wcat(params):
    beta_r = jnp.clip(jax.nn.softplus(params['beta_read']), 1.0, 20.0)
    beta_w = jnp.clip(jax.nn.softplus(params['beta_write']), 1.0, 20.0)
    C = HDIM + MDIM
    w = jnp.zeros((C, NCOL), jnp.float32)
    w = w.at[:, C_LOG:C_LOG + VOCAB].set(params['W_logits'])
    w = w.at[:, C_RK:C_RK + MHEADS * MDIM].set(params['W_rk'] * beta_r)
    w = w.at[:, C_WK:C_WK + MDIM].set(params['W_wk'] * beta_w)
    w = w.at[:, C_WV:C_WV + MDIM].set(params['W_wv'])
    w = w.at[:, C_ER:C_ER + MDIM].set(params['W_er'])
    w = w.at[:, C_AG:C_AG + 1].set(params['W_ag'])
    bias = jnp.zeros((1, NCOL), jnp.float32).at[0, :VOCAB].set(params['b_logits'])
    return w[:HDIM], w[HDIM:], bias


@functools.partial(jax.jit, static_argnames=('interpret',))
def kernel(input_seq, params, interpret=False):
    h = _controller_hidden(params, input_seq)
    h_tm = jnp.transpose(h, (1, 0, 2)).reshape(B * T, HDIM)
    w1, w2, bias = _build_wcat(params)
    dec = jax.nn.sigmoid(params['decay']).reshape(1, 1)

    out = pl.pallas_call(
        _scan_body,
        out_shape=jax.ShapeDtypeStruct((T, B, VOCAB), jnp.float32),
        scratch_shapes=[
            pltpu.VMEM((B, MDIM, SLOTS), jnp.float32),
            pltpu.VMEM((T, B, NCOL), jnp.float32),
        ],
        interpret=interpret,
    )(h_tm, w1, w2, bias, dec)
    return jnp.transpose(out, (1, 0, 2))


# speedup vs baseline: 16.9617x; 1.1077x over previous
import functools

import jax
import jax.numpy as jnp
import numpy as np
from jax.experimental import pallas as pl
from jax.experimental.pallas import tpu as pltpu

SLOTS = 1024
MDIM = 64
MHEADS = 4
TOPK = 32
VOCAB = 8192
EDIM = 512
HDIM = 512
NHATTN = 8
DFF = 2048
B = 4
T = 32

C_RK = 0
C_WK = 256
C_WV = 384
C_ER = 512
C_AG = 640
NCOL = 768
KEY_OFFS = (C_RK, C_RK + 64, C_RK + 128, C_RK + 192, C_WK)


def _scan_body(h_ref, wl_ref, ws_ref, bias_ref, dec_ref, out_ref,
               mem_ref, plog_ref, psm_ref):
    h = h_ref[...]
    plog_ref[...] = (jax.lax.dot_general(
        h, wl_ref[:HDIM], (((1,), (0,)), ((), ())),
        preferred_element_type=jnp.float32) + bias_ref[...]).reshape(T, B, VOCAB)
    psm_ref[...] = jax.lax.dot_general(
        h, ws_ref[:HDIM], (((1,), (0,)), ((), ())),
        preferred_element_type=jnp.float32).reshape(T, B, NCOL)
    mem_ref[...] = jnp.zeros((B, MDIM, SLOTS), jnp.float32)
    dec = dec_ref[...]
    wl2 = wl_ref[HDIM:]
    ws2 = ws_ref[HDIM:]

    iota = jax.lax.broadcasted_iota(jnp.int32, (5 * B, SLOTS), 1)

    def step(t, rv):
        out_ref[t] = plog_ref[t] + jax.lax.dot_general(
            rv, wl2, (((1,), (0,)), ((), ())), preferred_element_type=jnp.float32)
        pr = psm_ref[t] + jax.lax.dot_general(
            rv, ws2, (((1,), (0,)), ((), ())),
            preferred_element_type=jnp.float32)

        s_rows = []
        for b in range(B):
            kb = jnp.concatenate(
                [pr[b:b + 1, o:o + MDIM] for o in KEY_OFFS], axis=0)
            s_rows.append(jax.lax.dot_general(
                kb, mem_ref[b], (((1,), (0,)), ((), ())),
                preferred_element_type=jnp.float32))
        s_orig = jnp.concatenate(s_rows, axis=0)

        def extract(_, c):
            s, sel = c
            m = jnp.max(s, axis=1, keepdims=True)
            t_idx = jnp.min(jnp.where(s == m, iota, jnp.int32(2 ** 30)),
                            axis=1, keepdims=True)
            oh = iota == t_idx
            return jnp.where(oh, jnp.float32(-1e30), s), jnp.where(oh, 1.0, sel)

        _, sel = jax.lax.fori_loop(
            0, TOPK, extract, (s_orig, jnp.zeros((5 * B, SLOTS), jnp.float32)))

        gmax = jnp.max(s_orig, axis=1, keepdims=True)
        w_un = sel * jnp.exp(s_orig - gmax)
        wf = w_un / jnp.sum(w_un, axis=1, keepdims=True)

        wv_t = jnp.transpose(pr[:, C_WV:C_WV + MDIM])
        er_t = jnp.transpose(jax.nn.sigmoid(pr[:, C_ER:C_ER + MDIM]))
        ag = jax.nn.sigmoid(pr[:, C_AG:C_AG + 1])

        rv_rows = []
        for b in range(B):
            mb = mem_ref[b]
            wr = wf[5 * b:5 * b + MHEADS]
            rb = jax.lax.dot_general(
                wr, mb, (((1,), (1,)), ((), ())),
                preferred_element_type=jnp.float32)
            rv_rows.append(jnp.mean(rb, axis=0, keepdims=True))
            ww = wf[5 * b + MHEADS:5 * b + MHEADS + 1]
            upd = mb * (1.0 - er_t[:, b:b + 1] * ww) \
                + ag[b:b + 1, :] * wv_t[:, b:b + 1] * ww
            mem_ref[b] = upd * dec
        return jnp.concatenate(rv_rows, axis=0)

    jax.lax.fori_loop(0, T, step, jnp.zeros((B, MDIM), jnp.float32))


def _ln(x, g, b):
    m = jnp.mean(x, axis=-1, keepdims=True)
    v = jnp.var(x, axis=-1, keepdims=True)
    return (x - m) / jnp.sqrt(v + 1e-5) * g + b


def _controller_hidden(params, tokens):
    Bq, Tq = tokens.shape
    dh = HDIM // NHATTN
    x = params['embed'][tokens] + params['pos'][None, :Tq, :]
    x = x @ params['W_in']
    mask = jnp.tril(jnp.ones((Tq, Tq), dtype=bool))
    for L in params['layers']:
        h = _ln(x, L['ln1_g'], L['ln1_b'])
        q = (h @ L['Wq']).reshape(Bq, Tq, NHATTN, dh).transpose(0, 2, 1, 3)
        k = (h @ L['Wk']).reshape(Bq, Tq, NHATTN, dh).transpose(0, 2, 1, 3)
        v = (h @ L['Wv']).reshape(Bq, Tq, NHATTN, dh).transpose(0, 2, 1, 3)
        s = jnp.einsum('bhqd,bhkd->bhqk', q, k) / float(np.sqrt(dh))
        s = jnp.where(mask[None, None, :, :], s, -1e9)
        a = jax.nn.softmax(s, axis=-1)
        o = jnp.einsum('bhqk,bhkd->bhqd', a, v).transpose(0, 2, 1, 3).reshape(Bq, Tq, HDIM)
        x = x + o @ L['Wo']
        h2 = _ln(x, L['ln2_g'], L['ln2_b'])
        x = x + jax.nn.gelu(h2 @ L['W1'] + L['b1']) @ L['W2'] + L['b2']
    return _ln(x, params['lnf_g'], params['lnf_b'])


def _build_wsmall(params):
    beta_r = jnp.clip(jax.nn.softplus(params['beta_read']), 1.0, 20.0)
    beta_w = jnp.clip(jax.nn.softplus(params['beta_write']), 1.0, 20.0)
    C = HDIM + MDIM
    w = jnp.zeros((C, NCOL), jnp.float32)
    w = w.at[:, C_RK:C_RK + MHEADS * MDIM].set(params['W_rk'] * beta_r)
    w = w.at[:, C_WK:C_WK + MDIM].set(params['W_wk'] * beta_w)
    w = w.at[:, C_WV:C_WV + MDIM].set(params['W_wv'])
    w = w.at[:, C_ER:C_ER + MDIM].set(params['W_er'])
    w = w.at[:, C_AG:C_AG + 1].set(params['W_ag'])
    return w


@functools.partial(jax.jit, static_argnames=('interpret',))
def kernel(input_seq, params, interpret=False):
    h = _controller_hidden(params, input_seq)
    h_tm = jnp.transpose(h, (1, 0, 2)).reshape(B * T, HDIM)
    ws = _build_wsmall(params)
    bias = params['b_logits'].reshape(1, VOCAB)
    dec = jax.nn.sigmoid(params['decay']).reshape(1, 1)

    out = pl.pallas_call(
        _scan_body,
        out_shape=jax.ShapeDtypeStruct((T, B, VOCAB), jnp.float32),
        scratch_shapes=[
            pltpu.VMEM((B, MDIM, SLOTS), jnp.float32),
            pltpu.VMEM((T, B, VOCAB), jnp.float32),
            pltpu.VMEM((T, B, NCOL), jnp.float32),
        ],
        interpret=interpret,
    )(h_tm, params['W_logits'], ws, bias, dec)
    return jnp.transpose(out, (1, 0, 2))
